# baseline (speedup 1.0000x reference)
import jax
import jax.numpy as jnp
from jax.experimental import pallas as pl
from jax.experimental.pallas import tpu as pltpu
from jax.experimental.pallas import tpu_sc as plsc

B, C, H, W = 8, 96, 224, 224
HW = H * W
N = B * HW
EPS = 1e-4
CHUNK = 6272
NCHUNK = HW // CHUNK


def _stats_body(x_ref, s1_ref, s2_ref):
    xb = x_ref[0]
    ps = jnp.sum(xb, axis=1, keepdims=True)
    pq = jnp.sum(xb * xb, axis=1, keepdims=True)
    first = pl.program_id(1) == 0

    @pl.when(first)
    def _():
        s1_ref[0] = ps
        s2_ref[0] = pq

    @pl.when(jnp.logical_not(first))
    def _():
        s1_ref[0] += ps
        s2_ref[0] += pq


def _apply_body(x_ref, s1_ref, s2_ref, g0_ref, g1_ref, o_ref):
    inv_n = jnp.float32(1.0 / N)
    mean = jnp.sum(s1_ref[...], axis=0) * inv_n
    var = jnp.sum(s2_ref[...], axis=0) * inv_n - mean * mean
    inv = jax.lax.rsqrt(var + EPS)
    a = inv * (1.0 + g0_ref[0])
    c = g1_ref[0] - mean * a
    o_ref[...] = x_ref[...] * a[None] + c[None]


def _sc_gather(y2, table0, table1):
    mesh = plsc.VectorSubcoreMesh(core_axis_name="c", subcore_axis_name="s")
    cp = table0.shape[1]
    out_t = jax.ShapeDtypeStruct((B, cp), table0.dtype)

    @pl.kernel(out_type=(out_t, out_t), mesh=mesh)
    def k(t0_hbm, t1_hbm, y_hbm, o0_hbm, o1_hbm):
        def body(i_vmem, o0_vmem, o1_vmem):
            pltpu.sync_copy(t0_hbm.at[i_vmem.at[0]], o0_vmem)
            pltpu.sync_copy(t1_hbm.at[i_vmem.at[0]], o1_vmem)

        pltpu.emit_pipeline(
            body,
            grid=(1,),
            in_specs=[pl.BlockSpec((1, B), lambda i: (0, 0))],
            out_specs=[pl.BlockSpec((B, cp), lambda i: (0, 0)),
                       pl.BlockSpec((B, cp), lambda i: (0, 0))],
            core_axis_name="s",
            dimension_semantics=(pltpu.PARALLEL,),
        )(y_hbm, o0_hbm, o1_hbm)

    return k(table0, table1, y2)


def kernel(x, y, embed0, embed1):
    xv = x.reshape(B, C, HW)
    pad = ((0, 0), (0, 128 - C))
    e0y, e1y = _sc_gather(y.reshape(1, B),
                          jnp.pad(embed0, pad), jnp.pad(embed1, pad))
    g0 = e0y[:, :C].reshape(B, C, 1)
    g1 = e1y[:, :C].reshape(B, C, 1)

    s1, s2 = pl.pallas_call(
        _stats_body,
        grid=(B, NCHUNK),
        in_specs=[pl.BlockSpec((1, C, CHUNK), lambda b, j: (b, 0, j))],
        out_specs=[pl.BlockSpec((1, C, 1), lambda b, j: (b, 0, 0)),
                   pl.BlockSpec((1, C, 1), lambda b, j: (b, 0, 0))],
        out_shape=[jax.ShapeDtypeStruct((B, C, 1), jnp.float32)] * 2,
        compiler_params=pltpu.CompilerParams(
            dimension_semantics=("parallel", "arbitrary")),
    )(xv)

    out = pl.pallas_call(
        _apply_body,
        grid=(B, NCHUNK),
        in_specs=[pl.BlockSpec((1, C, CHUNK), lambda b, j: (b, 0, j)),
                  pl.BlockSpec((B, C, 1), lambda b, j: (0, 0, 0)),
                  pl.BlockSpec((B, C, 1), lambda b, j: (0, 0, 0)),
                  pl.BlockSpec((1, C, 1), lambda b, j: (b, 0, 0)),
                  pl.BlockSpec((1, C, 1), lambda b, j: (b, 0, 0))],
        out_specs=pl.BlockSpec((1, C, CHUNK), lambda b, j: (b, 0, j)),
        out_shape=jax.ShapeDtypeStruct((B, C, HW), jnp.float32),
        compiler_params=pltpu.CompilerParams(
            dimension_semantics=("parallel", "parallel")),
    )(xv, s1, s2, g0, g1)
    return out.reshape(B, C, H, W)


# speedup vs baseline: 1.0719x; 1.0719x over previous
import jax
import jax.numpy as jnp
from jax.experimental import pallas as pl
from jax.experimental.pallas import tpu as pltpu
from jax.experimental.pallas import tpu_sc as plsc

B, C, H, W = 8, 96, 224, 224
HW = H * W
N = B * HW
EPS = 1e-4
CHUNK = 6272
NCHUNK = HW // CHUNK
TILES = B * NCHUNK
NBUF = 8


def _fused_body(g0_ref, g1_ref, x_hbm, o_hbm,
                inb, outb, ab, cb, insem, outsem):
    def in_copy(t, slot):
        b = t // NCHUNK
        j = t % NCHUNK
        return pltpu.make_async_copy(
            x_hbm.at[b, :, pl.ds(j * CHUNK, CHUNK)], inb.at[slot],
            insem.at[slot])

    def out_copy(t, slot):
        b = t // NCHUNK
        j = t % NCHUNK
        return pltpu.make_async_copy(
            outb.at[slot], o_hbm.at[b, :, pl.ds(j * CHUNK, CHUNK)],
            outsem.at[slot])

    for t in range(NBUF):
        in_copy(t, t).start()

    def body1(t, carry):
        ps, pq = carry
        slot = jax.lax.rem(t, NBUF)
        in_copy(t, slot).wait()
        xb = inb[slot]
        ps = ps + jnp.sum(xb, axis=1, keepdims=True)
        pq = pq + jnp.sum(xb * xb, axis=1, keepdims=True)

        @pl.when(t + NBUF < TILES)
        def _():
            in_copy(t + NBUF, slot).start()
        return ps, pq

    zero = jnp.zeros((C, 1), jnp.float32)
    ps, pq = jax.lax.fori_loop(0, TILES, body1, (zero, zero))

    inv_n = jnp.float32(1.0 / N)
    mean = ps * inv_n
    var = pq * inv_n - mean * mean
    inv = jax.lax.rsqrt(var + EPS)
    a_all = inv[None] * (1.0 + g0_ref[...])
    ab[...] = a_all
    cb[...] = g1_ref[...] - mean[None] * a_all

    for t in range(NBUF):
        in_copy(t, t).start()

    def body2(t, _):
        slot = jax.lax.rem(t, NBUF)
        b = t // NCHUNK
        in_copy(t, slot).wait()

        @pl.when(t >= NBUF)
        def _():
            out_copy(t - NBUF, slot).wait()

        outb[slot] = inb[slot] * ab[b] + cb[b]
        out_copy(t, slot).start()

        @pl.when(t + NBUF < TILES)
        def _():
            in_copy(t + NBUF, slot).start()
        return 0

    jax.lax.fori_loop(0, TILES, body2, 0)
    for t in range(TILES - NBUF, TILES):
        out_copy(t, t % NBUF).wait()


def _sc_gather(y2, table0, table1):
    mesh = plsc.VectorSubcoreMesh(core_axis_name="c", subcore_axis_name="s")
    cp = table0.shape[1]
    out_t = jax.ShapeDtypeStruct((B, cp), table0.dtype)

    @pl.kernel(out_type=(out_t, out_t), mesh=mesh)
    def k(t0_hbm, t1_hbm, y_hbm, o0_hbm, o1_hbm):
        def body(i_vmem, o0_vmem, o1_vmem):
            pltpu.sync_copy(t0_hbm.at[i_vmem.at[0]], o0_vmem)
            pltpu.sync_copy(t1_hbm.at[i_vmem.at[0]], o1_vmem)

        pltpu.emit_pipeline(
            body,
            grid=(1,),
            in_specs=[pl.BlockSpec((1, B), lambda i: (0, 0))],
            out_specs=[pl.BlockSpec((B, cp), lambda i: (0, 0)),
                       pl.BlockSpec((B, cp), lambda i: (0, 0))],
            core_axis_name="s",
            dimension_semantics=(pltpu.PARALLEL,),
        )(y_hbm, o0_hbm, o1_hbm)

    return k(table0, table1, y2)


def kernel(x, y, embed0, embed1):
    xv = x.reshape(B, C, HW)
    pad = ((0, 0), (0, 128 - C))
    e0y, e1y = _sc_gather(y.reshape(1, B),
                          jnp.pad(embed0, pad), jnp.pad(embed1, pad))
    g0 = e0y[:, :C].reshape(B, C, 1)
    g1 = e1y[:, :C].reshape(B, C, 1)

    out = pl.pallas_call(
        _fused_body,
        in_specs=[pl.BlockSpec(memory_space=pltpu.MemorySpace.VMEM),
                  pl.BlockSpec(memory_space=pltpu.MemorySpace.VMEM),
                  pl.BlockSpec(memory_space=pl.ANY)],
        out_specs=pl.BlockSpec(memory_space=pl.ANY),
        out_shape=jax.ShapeDtypeStruct((B, C, HW), jnp.float32),
        scratch_shapes=[pltpu.MemorySpace.VMEM((NBUF, C, CHUNK), jnp.float32),
                        pltpu.MemorySpace.VMEM((NBUF, C, CHUNK), jnp.float32),
                        pltpu.MemorySpace.VMEM((B, C, 1), jnp.float32),
                        pltpu.MemorySpace.VMEM((B, C, 1), jnp.float32),
                        pltpu.SemaphoreType.DMA((NBUF,)),
                        pltpu.SemaphoreType.DMA((NBUF,))],
    )(g0, g1, xv)
    return out.reshape(B, C, H, W)


# speedup vs baseline: 2.7426x; 2.5587x over previous
import jax
import jax.numpy as jnp
from jax.experimental import pallas as pl
from jax.experimental.pallas import tpu as pltpu
from jax.experimental.pallas import tpu_sc as plsc

B, C, H, W = 8, 96, 224, 224
N = B * H * W
EPS = 1e-4
CB = 12
NCG = C // CB
TILES = B * NCG
NBUF = 8


def _fused_body(g0_ref, g1_ref, x_hbm, o_hbm,
                inb, outb, s1_ref, s2_ref, ab, cb_, insem, outsem):
    def in_copy(t, slot):
        b = t // NCG
        cg = t % NCG
        return pltpu.make_async_copy(
            x_hbm.at[b, pl.ds(cg * CB, CB)], inb.at[slot], insem.at[slot])

    def out_copy(t, slot):
        b = t // NCG
        cg = t % NCG
        return pltpu.make_async_copy(
            outb.at[slot], o_hbm.at[b, pl.ds(cg * CB, CB)], outsem.at[slot])

    for t in range(NBUF):
        in_copy(t, t).start()

    def body1(t, _):
        slot = jax.lax.rem(t, NBUF)
        b = t // NCG
        cg = jax.lax.rem(t, NCG)
        in_copy(t, slot).wait()
        xb = inb[slot]
        ps = jnp.sum(xb, axis=(1, 2), keepdims=True)
        pq = jnp.sum(xb * xb, axis=(1, 2), keepdims=True)

        @pl.when(b == 0)
        def _():
            s1_ref[cg] = ps
            s2_ref[cg] = pq

        @pl.when(b > 0)
        def _():
            s1_ref[cg] += ps
            s2_ref[cg] += pq

        @pl.when(t + NBUF < TILES)
        def _():
            in_copy(t + NBUF, slot).start()
        return 0

    jax.lax.fori_loop(0, TILES, body1, 0)

    inv_n = jnp.float32(1.0 / N)
    mean = s1_ref[...] * inv_n
    var = s2_ref[...] * inv_n - mean * mean
    inv = jax.lax.rsqrt(var + EPS)
    a_all = inv[None] * (1.0 + g0_ref[...])
    ab[...] = a_all
    cb_[...] = g1_ref[...] - mean[None] * a_all

    for t in range(NBUF):
        in_copy(t, t).start()

    def body2(t, _):
        slot = jax.lax.rem(t, NBUF)
        b = t // NCG
        cg = jax.lax.rem(t, NCG)
        in_copy(t, slot).wait()

        @pl.when(t >= NBUF)
        def _():
            out_copy(t - NBUF, slot).wait()

        outb[slot] = inb[slot] * ab[b, cg] + cb_[b, cg]
        out_copy(t, slot).start()

        @pl.when(t + NBUF < TILES)
        def _():
            in_copy(t + NBUF, slot).start()
        return 0

    jax.lax.fori_loop(0, TILES, body2, 0)
    for t in range(TILES - NBUF, TILES):
        out_copy(t, t % NBUF).wait()


def _sc_gather(y2, table0, table1):
    mesh = plsc.VectorSubcoreMesh(core_axis_name="c", subcore_axis_name="s")
    cp = table0.shape[1]
    out_t = jax.ShapeDtypeStruct((B, cp), table0.dtype)

    @pl.kernel(out_type=(out_t, out_t), mesh=mesh)
    def k(t0_hbm, t1_hbm, y_hbm, o0_hbm, o1_hbm):
        def body(i_vmem, o0_vmem, o1_vmem):
            pltpu.sync_copy(t0_hbm.at[i_vmem.at[0]], o0_vmem)
            pltpu.sync_copy(t1_hbm.at[i_vmem.at[0]], o1_vmem)

        pltpu.emit_pipeline(
            body,
            grid=(1,),
            in_specs=[pl.BlockSpec((1, B), lambda i: (0, 0))],
            out_specs=[pl.BlockSpec((B, cp), lambda i: (0, 0)),
                       pl.BlockSpec((B, cp), lambda i: (0, 0))],
            core_axis_name="s",
            dimension_semantics=(pltpu.PARALLEL,),
        )(y_hbm, o0_hbm, o1_hbm)

    return k(table0, table1, y2)


def kernel(x, y, embed0, embed1):
    pad = ((0, 0), (0, 128 - C))
    e0y, e1y = _sc_gather(y.reshape(1, B),
                          jnp.pad(embed0, pad), jnp.pad(embed1, pad))
    g0 = e0y[:, :C].reshape(B, NCG, CB, 1, 1)
    g1 = e1y[:, :C].reshape(B, NCG, CB, 1, 1)

    vmem = pltpu.MemorySpace.VMEM
    return pl.pallas_call(
        _fused_body,
        in_specs=[pl.BlockSpec(memory_space=vmem),
                  pl.BlockSpec(memory_space=vmem),
                  pl.BlockSpec(memory_space=pl.ANY)],
        out_specs=pl.BlockSpec(memory_space=pl.ANY),
        out_shape=jax.ShapeDtypeStruct((B, C, H, W), jnp.float32),
        scratch_shapes=[vmem((NBUF, CB, H, W), jnp.float32),
                        vmem((NBUF, CB, H, W), jnp.float32),
                        vmem((NCG, CB, 1, 1), jnp.float32),
                        vmem((NCG, CB, 1, 1), jnp.float32),
                        vmem((B, NCG, CB, 1, 1), jnp.float32),
                        vmem((B, NCG, CB, 1, 1), jnp.float32),
                        pltpu.SemaphoreType.DMA((NBUF,)),
                        pltpu.SemaphoreType.DMA((NBUF,))],
    )(g0, g1, x)


# speedup vs baseline: 2.7745x; 1.0116x over previous
import jax
import jax.numpy as jnp
from jax.experimental import pallas as pl
from jax.experimental.pallas import tpu as pltpu
from jax.experimental.pallas import tpu_sc as plsc

B, C, H, W = 8, 96, 224, 224
N = B * H * W
EPS = 1e-4
CB = 12
NCG = C // CB
TILES = B * NCG
NBUF = 8


def _fused_body(g0_ref, g1_ref, x_hbm, o_hbm,
                inb, outb, s1_ref, s2_ref, ab, cb_, insem, outsem):
    def in_copy(t, slot):
        b = t // NCG
        cg = t % NCG
        return pltpu.make_async_copy(
            x_hbm.at[b, pl.ds(cg * CB, CB)], inb.at[slot], insem.at[slot])

    def out_copy(t, slot):
        b = t // NCG
        cg = t % NCG
        return pltpu.make_async_copy(
            outb.at[slot], o_hbm.at[b, pl.ds(cg * CB, CB)], outsem.at[slot])

    for t in range(NBUF):
        in_copy(t, t).start()

    def body1(t, _):
        slot = jax.lax.rem(t, NBUF)
        b = t // NCG
        cg = jax.lax.rem(t, NCG)
        in_copy(t, slot).wait()
        xb = inb[slot]
        ps = jnp.sum(xb, axis=(1, 2), keepdims=True)
        pq = jnp.sum(xb * xb, axis=(1, 2), keepdims=True)

        @pl.when(b == 0)
        def _():
            s1_ref[cg] = ps
            s2_ref[cg] = pq

        @pl.when(b > 0)
        def _():
            s1_ref[cg] += ps
            s2_ref[cg] += pq

        in_copy(jax.lax.rem(t + NBUF, TILES), slot).start()
        return 0

    jax.lax.fori_loop(0, TILES, body1, 0)

    inv_n = jnp.float32(1.0 / N)
    mean = s1_ref[...] * inv_n
    var = s2_ref[...] * inv_n - mean * mean
    inv = jax.lax.rsqrt(var + EPS)
    a_all = inv[None] * (1.0 + g0_ref[...])
    ab[...] = a_all
    cb_[...] = g1_ref[...] - mean[None] * a_all

    def body2(t, _):
        slot = jax.lax.rem(t, NBUF)
        b = t // NCG
        cg = jax.lax.rem(t, NCG)
        in_copy(t, slot).wait()

        @pl.when(t >= NBUF)
        def _():
            out_copy(t - NBUF, slot).wait()

        outb[slot] = inb[slot] * ab[b, cg] + cb_[b, cg]
        out_copy(t, slot).start()

        @pl.when(t + NBUF < TILES)
        def _():
            in_copy(t + NBUF, slot).start()
        return 0

    jax.lax.fori_loop(0, TILES, body2, 0)
    for t in range(TILES - NBUF, TILES):
        out_copy(t, t % NBUF).wait()


def _sc_gather(y2, table0, table1):
    mesh = plsc.VectorSubcoreMesh(core_axis_name="c", subcore_axis_name="s")
    cp = table0.shape[1]
    out_t = jax.ShapeDtypeStruct((B, cp), table0.dtype)

    @pl.kernel(out_type=(out_t, out_t), mesh=mesh)
    def k(t0_hbm, t1_hbm, y_hbm, o0_hbm, o1_hbm):
        def body(i_vmem, o0_vmem, o1_vmem):
            pltpu.sync_copy(t0_hbm.at[i_vmem.at[0]], o0_vmem)
            pltpu.sync_copy(t1_hbm.at[i_vmem.at[0]], o1_vmem)

        pltpu.emit_pipeline(
            body,
            grid=(1,),
            in_specs=[pl.BlockSpec((1, B), lambda i: (0, 0))],
            out_specs=[pl.BlockSpec((B, cp), lambda i: (0, 0)),
                       pl.BlockSpec((B, cp), lambda i: (0, 0))],
            core_axis_name="s",
            dimension_semantics=(pltpu.PARALLEL,),
        )(y_hbm, o0_hbm, o1_hbm)

    return k(table0, table1, y2)


def kernel(x, y, embed0, embed1):
    pad = ((0, 0), (0, 128 - C))
    e0y, e1y = _sc_gather(y.reshape(1, B),
                          jnp.pad(embed0, pad), jnp.pad(embed1, pad))
    g0 = e0y[:, :C].reshape(B, NCG, CB, 1, 1)
    g1 = e1y[:, :C].reshape(B, NCG, CB, 1, 1)

    vmem = pltpu.MemorySpace.VMEM
    return pl.pallas_call(
        _fused_body,
        in_specs=[pl.BlockSpec(memory_space=vmem),
                  pl.BlockSpec(memory_space=vmem),
                  pl.BlockSpec(memory_space=pl.ANY)],
        out_specs=pl.BlockSpec(memory_space=pl.ANY),
        out_shape=jax.ShapeDtypeStruct((B, C, H, W), jnp.float32),
        scratch_shapes=[vmem((NBUF, CB, H, W), jnp.float32),
                        vmem((NBUF, CB, H, W), jnp.float32),
                        vmem((NCG, CB, 1, 1), jnp.float32),
                        vmem((NCG, CB, 1, 1), jnp.float32),
                        vmem((B, NCG, CB, 1, 1), jnp.float32),
                        vmem((B, NCG, CB, 1, 1), jnp.float32),
                        pltpu.SemaphoreType.DMA((NBUF,)),
                        pltpu.SemaphoreType.DMA((NBUF,))],
    )(g0, g1, x)


# speedup vs baseline: 2.7893x; 1.0053x over previous
import jax
import jax.numpy as jnp
from jax.experimental import pallas as pl
from jax.experimental.pallas import tpu as pltpu
from jax.experimental.pallas import tpu_sc as plsc

B, C, H, W = 8, 96, 224, 224
N = B * H * W
EPS = 1e-4
CB = 24
NCG = C // CB
TILES = B * NCG
NBUF = 4


def _fused_body(g0_ref, g1_ref, x_hbm, o_hbm,
                inb, outb, s1_ref, s2_ref, ab, cb_, insem, outsem):
    def in_copy(t, slot):
        b = t // NCG
        cg = t % NCG
        return pltpu.make_async_copy(
            x_hbm.at[b, pl.ds(cg * CB, CB)], inb.at[slot], insem.at[slot])

    def out_copy(t, slot):
        b = t // NCG
        cg = t % NCG
        return pltpu.make_async_copy(
            outb.at[slot], o_hbm.at[b, pl.ds(cg * CB, CB)], outsem.at[slot])

    for t in range(NBUF):
        in_copy(t, t).start()

    def body1(t, _):
        slot = jax.lax.rem(t, NBUF)
        b = t // NCG
        cg = jax.lax.rem(t, NCG)
        in_copy(t, slot).wait()
        xb = inb[slot]
        ps = jnp.sum(xb, axis=(1, 2), keepdims=True)
        pq = jnp.sum(xb * xb, axis=(1, 2), keepdims=True)

        @pl.when(b == 0)
        def _():
            s1_ref[cg] = ps
            s2_ref[cg] = pq

        @pl.when(b > 0)
        def _():
            s1_ref[cg] += ps
            s2_ref[cg] += pq

        in_copy(jax.lax.rem(t + NBUF, TILES), slot).start()
        return 0

    jax.lax.fori_loop(0, TILES, body1, 0)

    inv_n = jnp.float32(1.0 / N)
    mean = s1_ref[...] * inv_n
    var = s2_ref[...] * inv_n - mean * mean
    inv = jax.lax.rsqrt(var + EPS)
    a_all = inv[None] * (1.0 + g0_ref[...])
    ab[...] = a_all
    cb_[...] = g1_ref[...] - mean[None] * a_all

    def body2(t, _):
        slot = jax.lax.rem(t, NBUF)
        b = t // NCG
        cg = jax.lax.rem(t, NCG)
        in_copy(t, slot).wait()

        @pl.when(t >= NBUF)
        def _():
            out_copy(t - NBUF, slot).wait()

        outb[slot] = inb[slot] * ab[b, cg] + cb_[b, cg]
        out_copy(t, slot).start()

        @pl.when(t + NBUF < TILES)
        def _():
            in_copy(t + NBUF, slot).start()
        return 0

    jax.lax.fori_loop(0, TILES, body2, 0)
    for t in range(TILES - NBUF, TILES):
        out_copy(t, t % NBUF).wait()


def _sc_gather(y2, table0, table1):
    mesh = plsc.VectorSubcoreMesh(core_axis_name="c", subcore_axis_name="s")
    cp = table0.shape[1]
    out_t = jax.ShapeDtypeStruct((B, cp), table0.dtype)

    @pl.kernel(out_type=(out_t, out_t), mesh=mesh)
    def k(t0_hbm, t1_hbm, y_hbm, o0_hbm, o1_hbm):
        def body(i_vmem, o0_vmem, o1_vmem):
            pltpu.sync_copy(t0_hbm.at[i_vmem.at[0]], o0_vmem)
            pltpu.sync_copy(t1_hbm.at[i_vmem.at[0]], o1_vmem)

        pltpu.emit_pipeline(
            body,
            grid=(1,),
            in_specs=[pl.BlockSpec((1, B), lambda i: (0, 0))],
            out_specs=[pl.BlockSpec((B, cp), lambda i: (0, 0)),
                       pl.BlockSpec((B, cp), lambda i: (0, 0))],
            core_axis_name="s",
            dimension_semantics=(pltpu.PARALLEL,),
        )(y_hbm, o0_hbm, o1_hbm)

    return k(table0, table1, y2)


def kernel(x, y, embed0, embed1):
    pad = ((0, 0), (0, 128 - C))
    e0y, e1y = _sc_gather(y.reshape(1, B),
                          jnp.pad(embed0, pad), jnp.pad(embed1, pad))
    g0 = e0y[:, :C].reshape(B, NCG, CB, 1, 1)
    g1 = e1y[:, :C].reshape(B, NCG, CB, 1, 1)

    vmem = pltpu.MemorySpace.VMEM
    return pl.pallas_call(
        _fused_body,
        in_specs=[pl.BlockSpec(memory_space=vmem),
                  pl.BlockSpec(memory_space=vmem),
                  pl.BlockSpec(memory_space=pl.ANY)],
        out_specs=pl.BlockSpec(memory_space=pl.ANY),
        out_shape=jax.ShapeDtypeStruct((B, C, H, W), jnp.float32),
        scratch_shapes=[vmem((NBUF, CB, H, W), jnp.float32),
                        vmem((NBUF, CB, H, W), jnp.float32),
                        vmem((NCG, CB, 1, 1), jnp.float32),
                        vmem((NCG, CB, 1, 1), jnp.float32),
                        vmem((B, NCG, CB, 1, 1), jnp.float32),
                        vmem((B, NCG, CB, 1, 1), jnp.float32),
                        pltpu.SemaphoreType.DMA((NBUF,)),
                        pltpu.SemaphoreType.DMA((NBUF,))],
    )(g0, g1, x)


# speedup vs baseline: 3.8641x; 1.3853x over previous
import jax
import jax.numpy as jnp
from jax.experimental import pallas as pl
from jax.experimental.pallas import tpu as pltpu
from jax.experimental.pallas import tpu_sc as plsc

B, C, H, W = 8, 96, 224, 224
N = B * H * W
EPS = 1e-4
CB = 8
NCG = C // CB
NBUF = 4
assert B % NBUF == 0


def _fused_body(g0_ref, g1_ref, x_hbm, o_hbm,
                inb, outb, insem, outsem):
    def in_copy(g, b, buf):
        return pltpu.make_async_copy(
            x_hbm.at[b, pl.ds(g * CB, CB)], inb.at[buf, b], insem.at[buf, b])

    def out_copy(g, b, oslot):
        return pltpu.make_async_copy(
            outb.at[oslot], o_hbm.at[b, pl.ds(g * CB, CB)], outsem.at[oslot])

    for g in range(2):
        for b in range(B):
            in_copy(g, b, g).start()

    for g in range(NCG):
        buf = g % 2
        for b in range(B):
            in_copy(g, b, buf).wait()

        s1 = jnp.zeros((CB,), jnp.float32)
        s2 = jnp.zeros((CB,), jnp.float32)
        for b in range(B):
            xb = inb[buf, b]
            s1 = s1 + jnp.sum(xb, axis=(1, 2))
            s2 = s2 + jnp.sum(xb * xb, axis=(1, 2))
        inv_n = jnp.float32(1.0 / N)
        mean = s1 * inv_n
        var = s2 * inv_n - mean * mean
        inv = jax.lax.rsqrt(var + EPS)
        a = inv[None] * (1.0 + g0_ref[g])
        c = g1_ref[g] - mean[None] * a
        a = a[:, :, None, None]
        c = c[:, :, None, None]

        for b in range(B):
            oslot = b % NBUF
            if b >= NBUF:
                out_copy(g, b - NBUF, oslot).wait()
            elif g > 0:
                out_copy(g - 1, b + B - NBUF, oslot).wait()
            outb[oslot] = inb[buf, b] * a[b] + c[b]
            out_copy(g, b, oslot).start()
            if g + 2 < NCG:
                in_copy(g + 2, b, buf).start()
    for b in range(B - NBUF, B):
        out_copy(NCG - 1, b, b % NBUF).wait()


def _sc_gather(y2, table0, table1):
    mesh = plsc.VectorSubcoreMesh(core_axis_name="c", subcore_axis_name="s")
    cp = table0.shape[1]
    out_t = jax.ShapeDtypeStruct((B, cp), table0.dtype)

    @pl.kernel(out_type=(out_t, out_t), mesh=mesh)
    def k(t0_hbm, t1_hbm, y_hbm, o0_hbm, o1_hbm):
        def body(i_vmem, o0_vmem, o1_vmem):
            pltpu.sync_copy(t0_hbm.at[i_vmem.at[0]], o0_vmem)
            pltpu.sync_copy(t1_hbm.at[i_vmem.at[0]], o1_vmem)

        pltpu.emit_pipeline(
            body,
            grid=(1,),
            in_specs=[pl.BlockSpec((1, B), lambda i: (0, 0))],
            out_specs=[pl.BlockSpec((B, cp), lambda i: (0, 0)),
                       pl.BlockSpec((B, cp), lambda i: (0, 0))],
            core_axis_name="s",
            dimension_semantics=(pltpu.PARALLEL,),
        )(y_hbm, o0_hbm, o1_hbm)

    return k(table0, table1, y2)


def kernel(x, y, embed0, embed1):
    pad = ((0, 0), (0, 128 - C))
    e0y, e1y = _sc_gather(y.reshape(1, B),
                          jnp.pad(embed0, pad), jnp.pad(embed1, pad))
    g0 = e0y[:, :C].reshape(B, NCG, CB).transpose(1, 0, 2)
    g1 = e1y[:, :C].reshape(B, NCG, CB).transpose(1, 0, 2)

    vmem = pltpu.MemorySpace.VMEM
    return pl.pallas_call(
        _fused_body,
        in_specs=[pl.BlockSpec(memory_space=vmem),
                  pl.BlockSpec(memory_space=vmem),
                  pl.BlockSpec(memory_space=pl.ANY)],
        out_specs=pl.BlockSpec(memory_space=pl.ANY),
        out_shape=jax.ShapeDtypeStruct((B, C, H, W), jnp.float32),
        scratch_shapes=[vmem((2, B, CB, H, W), jnp.float32),
                        vmem((NBUF, CB, H, W), jnp.float32),
                        pltpu.SemaphoreType.DMA((2, B)),
                        pltpu.SemaphoreType.DMA((NBUF,))],
    )(g0, g1, x)
